# Initial kernel scaffold; baseline (speedup 1.0000x reference)
import jax
import jax.numpy as jnp
from jax.experimental import pallas as pl


def kernel(z, emb_w):
    raise NotImplementedError("write your pallas kernel here")


# speedup vs baseline: 1.0399x; 1.0399x over previous
import functools

import jax
import jax.numpy as jnp
from jax import lax
from jax.experimental import pallas as pl
from jax.experimental.pallas import tpu as pltpu
from jax.experimental.pallas import tpu_sc as plsc

_N_E = 8192
_E_DIM = 64
_BETA = 0.25
_BM = 512
_NCHUNK = 4
_W = _N_E // _NCHUNK


def _sqe_body(emb_ref, sqe_ref):
    e = emb_ref[...]
    sqe_ref[...] = jnp.sum(e * e, axis=1)


def _sqe_call(emb):
    return pl.pallas_call(
        _sqe_body,
        in_specs=[pl.BlockSpec(memory_space=pltpu.VMEM)],
        out_specs=pl.BlockSpec(memory_space=pltpu.VMEM),
        out_shape=jax.ShapeDtypeStruct((_N_E,), jnp.float32),
    )(emb)


def _argmin_body(z_ref, emt_ref, sqe_ref, idx_ref):
    z = z_ref[...]
    sqz = jnp.sum(z * z, axis=1)
    e = emt_ref[...]
    m = jnp.dot(z, e, preferred_element_type=jnp.float32)
    d = (sqz[:, None] + sqe_ref[...][None, :]) - 2.0 * m
    iota = lax.broadcasted_iota(jnp.int32, (_BM, _W), 1)
    best = None
    carry = None
    for j in range(_NCHUNK):
        dj = d[:, j * _W:(j + 1) * _W]
        mj = jnp.min(dj, axis=1)
        aj = jnp.min(jnp.where(dj == mj[:, None], iota + j * _W, jnp.int32(2**30)),
                     axis=1)
        mj_bf = mj.astype(jnp.bfloat16).astype(jnp.float32)
        if j == 0:
            best, carry = aj, mj_bf
        else:
            upd = mj < carry
            best = jnp.where(upd, aj, best)
            carry = jnp.where(upd, mj_bf, carry)
    idx_ref[...] = best


def _argmin_call(z, emt, sqe):
    n_rows = z.shape[0]
    grid = n_rows // _BM
    return pl.pallas_call(
        _argmin_body,
        grid=(grid,),
        in_specs=[
            pl.BlockSpec((_BM, _E_DIM), lambda i: (i, 0)),
            pl.BlockSpec((_E_DIM, _N_E), lambda i: (0, 0)),
            pl.BlockSpec((_N_E,), lambda i: (0,)),
        ],
        out_specs=pl.BlockSpec((_BM,), lambda i: (i,)),
        out_shape=jax.ShapeDtypeStruct((n_rows,), jnp.int32),
    )(z, emt, sqe)


def _make_gather(n_rows):
    info = plsc.get_sparse_core_info()
    nc, ns = info.num_cores, info.num_subcores
    nw = nc * ns
    b_per_w = n_rows // nw
    mesh = plsc.VectorSubcoreMesh(core_axis_name="c", subcore_axis_name="s")

    @functools.partial(
        pl.kernel,
        mesh=mesh,
        compiler_params=pltpu.CompilerParams(use_tc_tiling_on_sc=False),
        out_type=jax.ShapeDtypeStruct((n_rows, _E_DIM), jnp.float32),
        scratch_types=[
            pltpu.VMEM((b_per_w,), jnp.int32),
            pltpu.VMEM((b_per_w, _E_DIM), jnp.float32),
            pltpu.SemaphoreType.DMA,
        ],
    )
    def gather_k(idx_hbm, table_hbm, out_hbm, idx_v, rows_v, sem):
        wid = lax.axis_index("s") * nc + lax.axis_index("c")
        base = wid * b_per_w
        pltpu.sync_copy(idx_hbm.at[pl.ds(base, b_per_w)], idx_v)
        pltpu.async_copy(table_hbm.at[idx_v], rows_v, sem).wait()
        pltpu.sync_copy(rows_v, out_hbm.at[pl.ds(base, b_per_w)])

    return gather_k


def _out_loss_body(z_ref, zq_ref, out_ref, loss_ref):
    z = z_ref[...]
    zq = zq_ref[...]
    diff = zq - z
    out_ref[...] = z + diff
    m = jnp.mean(diff * diff)
    loss_ref[0, 0] = _BETA * m + m


def _out_loss_call(z, z_q):
    return pl.pallas_call(
        _out_loss_body,
        in_specs=[
            pl.BlockSpec(memory_space=pltpu.VMEM),
            pl.BlockSpec(memory_space=pltpu.VMEM),
        ],
        out_specs=(
            pl.BlockSpec(memory_space=pltpu.VMEM),
            pl.BlockSpec(memory_space=pltpu.SMEM),
        ),
        out_shape=(
            jax.ShapeDtypeStruct(z.shape, jnp.float32),
            jax.ShapeDtypeStruct((1, 1), jnp.float32),
        ),
    )(z, z_q)


def kernel(z, emb_w):
    emt = jnp.transpose(emb_w)
    sqe = _sqe_call(emb_w)
    idx = _argmin_call(z, emt, sqe)
    z_q = _make_gather(z.shape[0])(idx, emb_w)
    z_q_out, loss = _out_loss_call(z, z_q)
    return z_q_out, jnp.reshape(loss, ())


# speedup vs baseline: 1.0568x; 1.0162x over previous
import functools

import jax
import jax.numpy as jnp
from jax import lax
from jax.experimental import pallas as pl
from jax.experimental.pallas import tpu as pltpu
from jax.experimental.pallas import tpu_sc as plsc

_N_E = 8192
_E_DIM = 64
_BETA = 0.25
_BM = 1024
_NCHUNK = 4
_W = _N_E // _NCHUNK


def _sqe_body(emb_ref, sqe_ref):
    e = emb_ref[...]
    sqe_ref[...] = jnp.sum(e * e, axis=1)


def _sqe_call(emb):
    return pl.pallas_call(
        _sqe_body,
        in_specs=[pl.BlockSpec(memory_space=pltpu.VMEM)],
        out_specs=pl.BlockSpec(memory_space=pltpu.VMEM),
        out_shape=jax.ShapeDtypeStruct((_N_E,), jnp.float32),
    )(emb)


def _argmin_body(z_ref, emt_ref, sqe_ref, idx_ref):
    z = z_ref[...]
    sqz = jnp.sum(z * z, axis=1)
    e = emt_ref[...]
    m = jnp.dot(z, e, preferred_element_type=jnp.float32)
    d = (sqz[:, None] + sqe_ref[...][None, :]) - 2.0 * m
    iota = lax.broadcasted_iota(jnp.int32, (_BM, _W), 1)
    best = None
    carry = None
    for j in range(_NCHUNK):
        dj = d[:, j * _W:(j + 1) * _W]
        mj = jnp.min(dj, axis=1)
        aj = jnp.min(jnp.where(dj == mj[:, None], iota + j * _W, jnp.int32(2**30)),
                     axis=1)
        mj_bf = mj.astype(jnp.bfloat16).astype(jnp.float32)
        if j == 0:
            best, carry = aj, mj_bf
        else:
            upd = mj < carry
            best = jnp.where(upd, aj, best)
            carry = jnp.where(upd, mj_bf, carry)
    idx_ref[...] = best


def _argmin_call(z, emt, sqe):
    n_rows = z.shape[0]
    grid = n_rows // _BM
    return pl.pallas_call(
        _argmin_body,
        grid=(grid,),
        in_specs=[
            pl.BlockSpec((_BM, _E_DIM), lambda i: (i, 0)),
            pl.BlockSpec((_E_DIM, _N_E), lambda i: (0, 0)),
            pl.BlockSpec((_N_E,), lambda i: (0,)),
        ],
        out_specs=pl.BlockSpec((_BM,), lambda i: (i,)),
        out_shape=jax.ShapeDtypeStruct((n_rows,), jnp.int32),
    )(z, emt, sqe)


def _make_gather(n_rows):
    info = plsc.get_sparse_core_info()
    nc, ns = info.num_cores, info.num_subcores
    nw = nc * ns
    b_per_w = n_rows // nw
    mesh = plsc.VectorSubcoreMesh(core_axis_name="c", subcore_axis_name="s")

    @functools.partial(
        pl.kernel,
        mesh=mesh,
        compiler_params=pltpu.CompilerParams(use_tc_tiling_on_sc=False),
        out_type=jax.ShapeDtypeStruct((n_rows, _E_DIM), jnp.float32),
        scratch_types=[
            pltpu.VMEM((b_per_w,), jnp.int32),
            pltpu.VMEM((b_per_w, _E_DIM), jnp.float32),
            pltpu.SemaphoreType.DMA,
        ],
    )
    def gather_k(idx_hbm, table_hbm, out_hbm, idx_v, rows_v, sem):
        wid = lax.axis_index("s") * nc + lax.axis_index("c")
        base = wid * b_per_w
        pltpu.sync_copy(idx_hbm.at[pl.ds(base, b_per_w)], idx_v)
        pltpu.async_copy(table_hbm.at[idx_v], rows_v, sem).wait()
        pltpu.sync_copy(rows_v, out_hbm.at[pl.ds(base, b_per_w)])

    return gather_k


def _out_loss_body(z_ref, zq_ref, out_ref, loss_ref):
    z = z_ref[...]
    zq = zq_ref[...]
    diff = zq - z
    out_ref[...] = z + diff
    m = jnp.mean(diff * diff)
    loss_ref[0, 0] = _BETA * m + m


def _out_loss_call(z, z_q):
    return pl.pallas_call(
        _out_loss_body,
        in_specs=[
            pl.BlockSpec(memory_space=pltpu.VMEM),
            pl.BlockSpec(memory_space=pltpu.VMEM),
        ],
        out_specs=(
            pl.BlockSpec(memory_space=pltpu.VMEM),
            pl.BlockSpec(memory_space=pltpu.SMEM),
        ),
        out_shape=(
            jax.ShapeDtypeStruct(z.shape, jnp.float32),
            jax.ShapeDtypeStruct((1, 1), jnp.float32),
        ),
    )(z, z_q)


def kernel(z, emb_w):
    emt = jnp.transpose(emb_w)
    sqe = _sqe_call(emb_w)
    idx = _argmin_call(z, emt, sqe)
    z_q = _make_gather(z.shape[0])(idx, emb_w)
    z_q_out, loss = _out_loss_call(z, z_q)
    return z_q_out, jnp.reshape(loss, ())
